# baseline (speedup 1.0000x reference)
import functools

import jax
import jax.numpy as jnp
from jax import lax
from jax.experimental import pallas as pl
from jax.experimental.pallas import tpu as pltpu
from jax.experimental.pallas import tpu_sc as plsc

N = 10000
NPAD = 10240
E = 320000
GPT = 160
NGRP = GPT * 16
EPAD = NGRP * 128
FH = 128
BM = 1280
GRID_M = NPAD // BM
RPT = NPAD // 16


@functools.cache
def _mesh():
    return plsc.VectorSubcoreMesh(core_axis_name="c", subcore_axis_name="s")


GPH = GPT // 2
DEG_LAG = 8


def _deg_body(dst_hbm, init_hbm, degp_hbm, didx_v, ones_v, z_sp, ssem):
    c = lax.axis_index("c")
    s = lax.axis_index("s")
    r0 = s * RPT
    ib = pl.multiple_of(c * RPT, 8)
    pltpu.sync_copy(init_hbm.at[pl.ds(ib, RPT)], z_sp.at[pl.ds(r0, RPT)])
    pltpu.sync_copy(init_hbm.at[pl.ds(0, 128)], ones_v)
    g0 = pl.multiple_of(c * (NGRP // 2) + s * GPH, 8)
    pltpu.sync_copy(dst_hbm.at[pl.ds(g0, GPH)], didx_v)
    plsc.subcore_barrier()

    def step(j, carry):
        pltpu.async_copy(ones_v, z_sp.at[didx_v.at[j]], ssem, add=True)

        @pl.when(j >= DEG_LAG)
        def _():
            pltpu.make_async_copy(ones_v, z_sp.at[didx_v.at[j - DEG_LAG]], ssem).wait()

        return carry

    lax.fori_loop(0, GPH, step, 0)

    def dstep(j, carry):
        pltpu.make_async_copy(ones_v, z_sp.at[didx_v.at[j]], ssem).wait()
        return carry

    lax.fori_loop(GPH - DEG_LAG, GPH, dstep, 0)
    plsc.subcore_barrier()
    pltpu.sync_copy(z_sp.at[pl.ds(r0, RPT)],
                    degp_hbm.at[pl.ds(pl.multiple_of(c * NPAD + r0, 8), RPT)])


@functools.cache
def _deg_kernel():
    return pl.kernel(
        _deg_body,
        out_type=jax.ShapeDtypeStruct((2 * NPAD, FH), jnp.float32),
        mesh=_mesh(),
        scratch_types=[
            pltpu.VMEM((GPH, 128), jnp.int32),
            pltpu.VMEM((128, FH), jnp.float32),
            pltpu.VMEM_SHARED((NPAD, FH), jnp.float32),
            pltpu.SemaphoreType.DMA,
        ],
    )


EG = 64
NG64 = EPAD // EG
GPT64 = NG64 // 16
GC = 16
PASS_SZ = (3584, 3584, 3072)
ZROWS = 3584 + EG
YROWS = 10112
RPT_Y = YROWS // 16


def _propagate_body(y_hbm, src_hbm, dstc_hbm, z_hbm, sidx_v, didx_v, rows_v,
                    y_sp, z_sp, gsem, ssem):
    c = lax.axis_index("c")
    s = lax.axis_index("s")
    r0 = s * RPT_Y
    yb = pl.multiple_of(c * NPAD + r0, 8)

    pltpu.sync_copy(y_hbm.at[pl.ds(yb, RPT_Y)], y_sp.at[pl.ds(r0, RPT_Y)])

    base = 0
    for p, sz in enumerate(PASS_SZ):
        szt = sz // 16
        zr0 = s * szt
        zb = pl.multiple_of(c * NPAD + base + zr0, 8)
        pltpu.sync_copy(y_hbm.at[pl.ds(zb, szt)], z_sp.at[pl.ds(zr0, szt)])
        plsc.subcore_barrier()

        def chunk(k, carry):
            g0 = pl.multiple_of(s * GPT64 + k * GC, 8)
            pltpu.sync_copy(src_hbm.at[pl.ds(g0, GC)], sidx_v)
            pltpu.sync_copy(dstc_hbm.at[p, pl.ds(g0, GC)], didx_v)

            pltpu.async_copy(y_sp.at[sidx_v.at[0]], rows_v.at[0], gsem)

            def step(j, carry2):
                cur = j % 2
                nxt = (j + 1) % 2
                pltpu.make_async_copy(
                    y_sp.at[sidx_v.at[j]], rows_v.at[cur], gsem).wait()

                @pl.when(j >= 1)
                def _():
                    pltpu.make_async_copy(
                        rows_v.at[nxt], z_sp.at[didx_v.at[j - 1]],
                        ssem.at[nxt]).wait()

                @pl.when(j + 1 < GC)
                def _():
                    pltpu.async_copy(y_sp.at[sidx_v.at[j + 1]], rows_v.at[nxt],
                                     gsem)

                pltpu.async_copy(rows_v.at[cur], z_sp.at[didx_v.at[j]],
                                 ssem.at[cur], add=True)
                return carry2

            lax.fori_loop(0, GC, step, carry)
            last = (GC - 1) % 2
            pltpu.make_async_copy(
                rows_v.at[last], z_sp.at[didx_v.at[GC - 1]], ssem.at[last]).wait()
            return carry

        lax.fori_loop(0, GPT64 // GC, chunk, 0)
        plsc.subcore_barrier()
        pltpu.sync_copy(z_sp.at[pl.ds(zr0, szt)], z_hbm.at[pl.ds(zb, szt)])
        plsc.subcore_barrier()
        base += sz


@functools.cache
def _propagate_kernel():
    return pl.kernel(
        _propagate_body,
        out_type=jax.ShapeDtypeStruct((2 * NPAD, FH), jnp.float32),
        mesh=_mesh(),
        scratch_types=[
            pltpu.VMEM((GC, EG), jnp.int32),
            pltpu.VMEM((GC, EG), jnp.int32),
            pltpu.VMEM((2, EG, FH), jnp.float32),
            pltpu.VMEM_SHARED((YROWS, FH), jnp.float32),
            pltpu.VMEM_SHARED((ZROWS, FH), jnp.float32),
            pltpu.SemaphoreType.DMA,
            pltpu.SemaphoreType.DMA((2,)),
        ],
    )


def _tca_body(x_ref, w_ref, deg0_ref, deg1_ref, y_ref, dinv_ref):
    dinv = lax.rsqrt(deg0_ref[:, :1] + deg1_ref[:, :1])
    y = jnp.dot(x_ref[...], w_ref[...], preferred_element_type=jnp.float32)
    y = y * dinv
    y_ref[0] = y[:, :FH]
    y_ref[1] = y[:, FH:]
    dinv_ref[...] = dinv


def _tcb_body(z_ref, dinv_ref, b_ref, w_ref, h_ref, y_ref):
    dinv = dinv_ref[...]
    h0 = jnp.maximum(z_ref[0] * dinv + b_ref[0], 0.0)
    h1 = jnp.maximum(z_ref[1] * dinv + b_ref[1], 0.0)
    h_ref[0] = h0
    h_ref[1] = h1
    hf = jnp.concatenate([h0, h1], axis=1)
    y = jnp.dot(hf, w_ref[...], preferred_element_type=jnp.float32) * dinv
    y_ref[0] = y[:, :FH]
    y_ref[1] = y[:, FH:]


def _tcc_body(z_ref, dinv_ref, b_ref, h1_ref, h2_ref, wout_ref, bout_ref, out_ref):
    dinv = dinv_ref[...]
    h30 = jnp.maximum(z_ref[0] * dinv + b_ref[0], 0.0)
    h31 = jnp.maximum(z_ref[1] * dinv + b_ref[1], 0.0)
    jk0 = jnp.maximum(jnp.maximum(h1_ref[0], h2_ref[0]), h30)
    jk1 = jnp.maximum(jnp.maximum(h1_ref[1], h2_ref[1]), h31)
    acc = jnp.sum(jk0 * wout_ref[0, :, 0][None, :], axis=1, keepdims=True)
    acc = acc + jnp.sum(jk1 * wout_ref[1, :, 0][None, :], axis=1, keepdims=True)
    out_ref[...] = acc + bout_ref[0, 0]


def _half_spec():
    return pl.BlockSpec((2, BM, FH), lambda i: (0, i, 0))


def _full_spec(shape):
    return pl.BlockSpec(shape, lambda i: (0,) * len(shape))


def _tca(x, w1, degp):
    return pl.pallas_call(
        _tca_body,
        grid=(GRID_M,),
        in_specs=[
            pl.BlockSpec((BM, 128), lambda i: (i, 0)),
            _full_spec((128, 256)),
            pl.BlockSpec((BM, FH), lambda i: (i, 0)),
            pl.BlockSpec((BM, FH), lambda i: (i + GRID_M, 0)),
        ],
        out_specs=[_half_spec(), pl.BlockSpec((BM, 1), lambda i: (i, 0))],
        out_shape=[
            jax.ShapeDtypeStruct((2, NPAD, FH), jnp.float32),
            jax.ShapeDtypeStruct((NPAD, 1), jnp.float32),
        ],
    )(x, w1, degp, degp)


def _tcb(z, dinv, b, w):
    return pl.pallas_call(
        _tcb_body,
        grid=(GRID_M,),
        in_specs=[
            _half_spec(),
            pl.BlockSpec((BM, 1), lambda i: (i, 0)),
            _full_spec((2, 1, FH)),
            _full_spec((256, 256)),
        ],
        out_specs=[_half_spec(), _half_spec()],
        out_shape=[
            jax.ShapeDtypeStruct((2, NPAD, FH), jnp.float32),
            jax.ShapeDtypeStruct((2, NPAD, FH), jnp.float32),
        ],
    )(z, dinv, b, w)


def _tcc(z, dinv, b, h1, h2, wout, bout):
    return pl.pallas_call(
        _tcc_body,
        grid=(GRID_M,),
        in_specs=[
            _half_spec(),
            pl.BlockSpec((BM, 1), lambda i: (i, 0)),
            _full_spec((2, 1, FH)),
            _half_spec(),
            _half_spec(),
            _full_spec((2, FH, 1)),
            _full_spec((1, 1)),
        ],
        out_specs=[pl.BlockSpec((BM, 1), lambda i: (i, 0))],
        out_shape=[jax.ShapeDtypeStruct((NPAD, 1), jnp.float32)],
    )(z, dinv, b, h1, h2, wout, bout)[0]


def kernel(x, edge_index, W1, b1, W2, b2, W3, b3, Wout, bout):
    src = edge_index[0].astype(jnp.int32)
    dst = edge_index[1].astype(jnp.int32)
    pad = jnp.full((EPAD - E,), N, jnp.int32)
    srcf = jnp.concatenate([src, pad])
    dstf = jnp.concatenate([dst, pad])
    dst_p = dstf.reshape(NGRP, 128)
    src64 = srcf.reshape(NG64, EG)
    dump = 3584 + (jnp.arange(EPAD, dtype=jnp.int32) % EG)
    passes = []
    b = 0
    for sz in PASS_SZ:
        loc = dstf - b
        passes.append(jnp.where((loc >= 0) & (loc < sz), loc, dump))
        b += sz
    dstc = jnp.stack(passes).reshape(3, NG64, EG)

    xp = jnp.zeros((NPAD, 128), jnp.float32).at[:N].set(x)

    prop = _propagate_kernel()

    deg_init = jnp.concatenate([
        jnp.ones((RPT, FH), jnp.float32),
        jnp.zeros((RPT, FH), jnp.float32),
    ])
    deg = _deg_kernel()(dst_p, deg_init)

    b1r = b1.reshape(2, 1, FH)
    b2r = b2.reshape(2, 1, FH)
    b3r = b3.reshape(2, 1, FH)
    woutr = Wout.reshape(2, FH, 1)
    boutr = bout.reshape(1, 1)

    y1, dinv = _tca(xp, W1, deg)
    z1 = prop(y1.reshape(2 * NPAD, FH), src64, dstc).reshape(2, NPAD, FH)
    h1, y2 = _tcb(z1, dinv, b1r, W2)
    z2 = prop(y2.reshape(2 * NPAD, FH), src64, dstc).reshape(2, NPAD, FH)
    h2, y3 = _tcb(z2, dinv, b2r, W3)
    z3 = prop(y3.reshape(2 * NPAD, FH), src64, dstc).reshape(2, NPAD, FH)
    out = _tcc(z3, dinv, b3r, h1, h2, woutr, boutr)
    return out[:N, 0]


# speedup vs baseline: 1.3240x; 1.3240x over previous
import functools

import jax
import jax.numpy as jnp
from jax import lax
from jax.experimental import pallas as pl
from jax.experimental.pallas import tpu as pltpu
from jax.experimental.pallas import tpu_sc as plsc

N = 10000
NPAD = 10240
E = 320000
GPT = 160
NGRP = GPT * 16
EPAD = NGRP * 128
FH = 128
BM = 1280
GRID_M = NPAD // BM
RPT = NPAD // 16


@functools.cache
def _mesh():
    return plsc.VectorSubcoreMesh(core_axis_name="c", subcore_axis_name="s")


GPH = GPT // 2
DEG_LAG = 8


def _deg_body(dst_hbm, init_hbm, degp_hbm, didx_v, ones_v, z_sp, ssem):
    c = lax.axis_index("c")
    s = lax.axis_index("s")
    r0 = s * RPT
    ib = pl.multiple_of(c * RPT, 8)
    pltpu.sync_copy(init_hbm.at[pl.ds(ib, RPT)], z_sp.at[pl.ds(r0, RPT)])
    pltpu.sync_copy(init_hbm.at[pl.ds(0, 128)], ones_v)
    g0 = pl.multiple_of(c * (NGRP // 2) + s * GPH, 8)
    pltpu.sync_copy(dst_hbm.at[pl.ds(g0, GPH)], didx_v)
    plsc.subcore_barrier()

    def step(j, carry):
        pltpu.async_copy(ones_v, z_sp.at[didx_v.at[j]], ssem, add=True)

        @pl.when(j >= DEG_LAG)
        def _():
            pltpu.make_async_copy(ones_v, z_sp.at[didx_v.at[j - DEG_LAG]], ssem).wait()

        return carry

    lax.fori_loop(0, GPH, step, 0)

    def dstep(j, carry):
        pltpu.make_async_copy(ones_v, z_sp.at[didx_v.at[j]], ssem).wait()
        return carry

    lax.fori_loop(GPH - DEG_LAG, GPH, dstep, 0)
    plsc.subcore_barrier()
    pltpu.sync_copy(z_sp.at[pl.ds(r0, RPT)],
                    degp_hbm.at[pl.ds(pl.multiple_of(c * NPAD + r0, 8), RPT)])


@functools.cache
def _deg_kernel():
    return pl.kernel(
        _deg_body,
        out_type=jax.ShapeDtypeStruct((2 * NPAD, FH), jnp.float32),
        mesh=_mesh(),
        scratch_types=[
            pltpu.VMEM((GPH, 128), jnp.int32),
            pltpu.VMEM((128, FH), jnp.float32),
            pltpu.VMEM_SHARED((NPAD, FH), jnp.float32),
            pltpu.SemaphoreType.DMA,
        ],
    )


GCH = 40


def _propagate_body(y_hbm, src_hbm, dst_hbm, z_hbm, sidx_v, didx_v, rows_v, z_sp,
                    gsem, ssem):
    c = lax.axis_index("c")
    s = lax.axis_index("s")
    r0 = s * RPT
    yb = pl.multiple_of(c * NPAD + r0, 8)

    pltpu.sync_copy(y_hbm.at[pl.ds(yb, RPT)], z_sp.at[pl.ds(r0, RPT)])
    plsc.subcore_barrier()

    def chunk(k, carry):
        g0 = pl.multiple_of(s * GPT + k * GCH, 8)
        pltpu.sync_copy(src_hbm.at[pl.ds(c, 1), pl.ds(g0, GCH)], sidx_v)
        pltpu.sync_copy(dst_hbm.at[pl.ds(g0, GCH)], didx_v)

        pltpu.async_copy(y_hbm.at[sidx_v.at[0, 0]], rows_v.at[0], gsem)

        def step(j, carry2):
            cur = j % 2
            nxt = (j + 1) % 2
            pltpu.make_async_copy(
                y_hbm.at[sidx_v.at[0, j]], rows_v.at[cur], gsem).wait()

            @pl.when(j >= 1)
            def _():
                pltpu.make_async_copy(
                    rows_v.at[nxt], z_sp.at[didx_v.at[j - 1]], ssem.at[nxt]).wait()

            @pl.when(j + 1 < GCH)
            def _():
                pltpu.async_copy(y_hbm.at[sidx_v.at[0, j + 1]], rows_v.at[nxt], gsem)

            pltpu.async_copy(rows_v.at[cur], z_sp.at[didx_v.at[j]],
                             ssem.at[cur], add=True)
            return carry2

        lax.fori_loop(0, GCH, step, carry)
        last = (GCH - 1) % 2
        pltpu.make_async_copy(
            rows_v.at[last], z_sp.at[didx_v.at[GCH - 1]], ssem.at[last]).wait()
        return carry

    lax.fori_loop(0, GPT // GCH, chunk, 0)
    plsc.subcore_barrier()
    pltpu.sync_copy(z_sp.at[pl.ds(r0, RPT)], z_hbm.at[pl.ds(yb, RPT)])


@functools.cache
def _propagate_kernel():
    return pl.kernel(
        _propagate_body,
        out_type=jax.ShapeDtypeStruct((2 * NPAD, FH), jnp.float32),
        mesh=_mesh(),
        scratch_types=[
            pltpu.VMEM((1, GCH, 128), jnp.int32),
            pltpu.VMEM((GCH, 128), jnp.int32),
            pltpu.VMEM((2, 128, FH), jnp.float32),
            pltpu.VMEM_SHARED((NPAD, FH), jnp.float32),
            pltpu.SemaphoreType.DMA,
            pltpu.SemaphoreType.DMA((2,)),
        ],
    )


def _tca_body(x_ref, w_ref, deg0_ref, deg1_ref, y_ref, dinv_ref):
    dinv = lax.rsqrt(deg0_ref[:, :1] + deg1_ref[:, :1])
    y = jnp.dot(x_ref[...], w_ref[...], preferred_element_type=jnp.float32)
    y = y * dinv
    y_ref[0] = y[:, :FH]
    y_ref[1] = y[:, FH:]
    dinv_ref[...] = dinv


def _tcb_body(z_ref, dinv_ref, b_ref, w_ref, h_ref, y_ref):
    dinv = dinv_ref[...]
    h0 = jnp.maximum(z_ref[0] * dinv + b_ref[0], 0.0)
    h1 = jnp.maximum(z_ref[1] * dinv + b_ref[1], 0.0)
    h_ref[0] = h0
    h_ref[1] = h1
    hf = jnp.concatenate([h0, h1], axis=1)
    y = jnp.dot(hf, w_ref[...], preferred_element_type=jnp.float32) * dinv
    y_ref[0] = y[:, :FH]
    y_ref[1] = y[:, FH:]


def _tcc_body(z_ref, dinv_ref, b_ref, h1_ref, h2_ref, wout_ref, bout_ref, out_ref):
    dinv = dinv_ref[...]
    h30 = jnp.maximum(z_ref[0] * dinv + b_ref[0], 0.0)
    h31 = jnp.maximum(z_ref[1] * dinv + b_ref[1], 0.0)
    jk0 = jnp.maximum(jnp.maximum(h1_ref[0], h2_ref[0]), h30)
    jk1 = jnp.maximum(jnp.maximum(h1_ref[1], h2_ref[1]), h31)
    acc = jnp.sum(jk0 * wout_ref[0, :, 0][None, :], axis=1, keepdims=True)
    acc = acc + jnp.sum(jk1 * wout_ref[1, :, 0][None, :], axis=1, keepdims=True)
    out_ref[...] = acc + bout_ref[0, 0]


def _half_spec():
    return pl.BlockSpec((2, BM, FH), lambda i: (0, i, 0))


def _full_spec(shape):
    return pl.BlockSpec(shape, lambda i: (0,) * len(shape))


def _tca(x, w1, degp):
    return pl.pallas_call(
        _tca_body,
        grid=(GRID_M,),
        in_specs=[
            pl.BlockSpec((BM, 128), lambda i: (i, 0)),
            _full_spec((128, 256)),
            pl.BlockSpec((BM, FH), lambda i: (i, 0)),
            pl.BlockSpec((BM, FH), lambda i: (i + GRID_M, 0)),
        ],
        out_specs=[_half_spec(), pl.BlockSpec((BM, 1), lambda i: (i, 0))],
        out_shape=[
            jax.ShapeDtypeStruct((2, NPAD, FH), jnp.float32),
            jax.ShapeDtypeStruct((NPAD, 1), jnp.float32),
        ],
    )(x, w1, degp, degp)


def _tcb(z, dinv, b, w):
    return pl.pallas_call(
        _tcb_body,
        grid=(GRID_M,),
        in_specs=[
            _half_spec(),
            pl.BlockSpec((BM, 1), lambda i: (i, 0)),
            _full_spec((2, 1, FH)),
            _full_spec((256, 256)),
        ],
        out_specs=[_half_spec(), _half_spec()],
        out_shape=[
            jax.ShapeDtypeStruct((2, NPAD, FH), jnp.float32),
            jax.ShapeDtypeStruct((2, NPAD, FH), jnp.float32),
        ],
    )(z, dinv, b, w)


def _tcc(z, dinv, b, h1, h2, wout, bout):
    return pl.pallas_call(
        _tcc_body,
        grid=(GRID_M,),
        in_specs=[
            _half_spec(),
            pl.BlockSpec((BM, 1), lambda i: (i, 0)),
            _full_spec((2, 1, FH)),
            _half_spec(),
            _half_spec(),
            _full_spec((2, FH, 1)),
            _full_spec((1, 1)),
        ],
        out_specs=[pl.BlockSpec((BM, 1), lambda i: (i, 0))],
        out_shape=[jax.ShapeDtypeStruct((NPAD, 1), jnp.float32)],
    )(z, dinv, b, h1, h2, wout, bout)[0]


def kernel(x, edge_index, W1, b1, W2, b2, W3, b3, Wout, bout):
    src = edge_index[0].astype(jnp.int32)
    dst = edge_index[1].astype(jnp.int32)
    pad = jnp.full((EPAD - E,), N, jnp.int32)
    src_p = jnp.concatenate([src, pad]).reshape(NGRP, 128)
    dst_p = jnp.concatenate([dst, pad]).reshape(NGRP, 128)
    src2 = jnp.stack([src_p, src_p + NPAD])

    xp = jnp.zeros((NPAD, 128), jnp.float32).at[:N].set(x)

    prop = _propagate_kernel()

    deg_init = jnp.concatenate([
        jnp.ones((RPT, FH), jnp.float32),
        jnp.zeros((RPT, FH), jnp.float32),
    ])
    deg = _deg_kernel()(dst_p, deg_init)

    b1r = b1.reshape(2, 1, FH)
    b2r = b2.reshape(2, 1, FH)
    b3r = b3.reshape(2, 1, FH)
    woutr = Wout.reshape(2, FH, 1)
    boutr = bout.reshape(1, 1)

    y1, dinv = _tca(xp, W1, deg)
    z1 = prop(y1.reshape(2 * NPAD, FH), src2, dst_p).reshape(2, NPAD, FH)
    h1, y2 = _tcb(z1, dinv, b1r, W2)
    z2 = prop(y2.reshape(2 * NPAD, FH), src2, dst_p).reshape(2, NPAD, FH)
    h2, y3 = _tcb(z2, dinv, b2r, W3)
    z3 = prop(y3.reshape(2 * NPAD, FH), src2, dst_p).reshape(2, NPAD, FH)
    out = _tcc(z3, dinv, b3r, h1, h2, woutr, boutr)
    return out[:N, 0]
